# Initial kernel scaffold; baseline (speedup 1.0000x reference)
import jax
import jax.numpy as jnp
from jax.experimental import pallas as pl


def kernel(query_repr, W1, b1, W2, b2):
    raise NotImplementedError("write your pallas kernel here")


# speedup vs baseline: 4.1841x; 4.1841x over previous
import jax
import jax.numpy as jnp
from jax.experimental import pallas as pl

B = 32768
H = 768
H2 = 384
E = 64
TB = 512


def _gating_kernel(x_ref, w1_ref, b1_ref, w2_ref, b2_ref, out_ref):
    x = x_ref[...]
    h = jnp.dot(x, w1_ref[...], preferred_element_type=jnp.float32)
    h = jnp.maximum(h + b1_ref[...], 0.0)
    logits = jnp.dot(h, w2_ref[...], preferred_element_type=jnp.float32)
    logits = logits + b2_ref[...]

    col = jax.lax.broadcasted_iota(jnp.int32, logits.shape, 1)
    m1 = jnp.max(logits, axis=1, keepdims=True)
    i1 = jnp.min(jnp.where(logits == m1, col, E), axis=1, keepdims=True)
    is1 = col == i1
    masked = jnp.where(is1, -jnp.inf, logits)
    m2 = jnp.max(masked, axis=1, keepdims=True)
    i2 = jnp.min(jnp.where(masked == m2, col, E), axis=1, keepdims=True)
    is2 = col == i2
    e2 = jnp.exp(m2 - m1)
    g1 = 1.0 / (1.0 + e2)
    g2 = e2 * g1
    out_ref[...] = jnp.where(is1, g1, jnp.where(is2, g2, 0.0))


def kernel(query_repr, W1, b1, W2, b2):
    b1r = b1.reshape(1, H2)
    b2r = b2.reshape(1, E)
    grid = (B // TB,)
    return pl.pallas_call(
        _gating_kernel,
        grid=grid,
        in_specs=[
            pl.BlockSpec((TB, H), lambda i: (i, 0)),
            pl.BlockSpec((H, H2), lambda i: (0, 0)),
            pl.BlockSpec((1, H2), lambda i: (0, 0)),
            pl.BlockSpec((H2, E), lambda i: (0, 0)),
            pl.BlockSpec((1, E), lambda i: (0, 0)),
        ],
        out_specs=pl.BlockSpec((TB, E), lambda i: (i, 0)),
        out_shape=jax.ShapeDtypeStruct((B, E), jnp.float32),
    )(query_repr, W1, b1r, W2, b2r)


# speedup vs baseline: 4.5219x; 1.0807x over previous
import jax
import jax.numpy as jnp
from jax.experimental import pallas as pl

B = 32768
H = 768
H2 = 384
E = 64
TB = 512


def _gating_kernel(x_ref, w1_ref, b1_ref, w2_ref, b2_ref, out_ref):
    x = x_ref[...]
    h = jnp.dot(x, w1_ref[...], preferred_element_type=jnp.float32)
    h = jnp.maximum(h + b1_ref[...], 0.0)
    logits = jnp.dot(h, w2_ref[...], preferred_element_type=jnp.float32)
    logits = logits + b2_ref[...]

    ncol = -jax.lax.broadcasted_iota(jnp.int32, logits.shape, 1).astype(jnp.float32)
    ninf = jnp.float32(-jnp.inf)
    m1 = jnp.max(logits, axis=1, keepdims=True)
    t1 = jnp.where(logits == m1, ncol, ninf)
    i1n = jnp.max(t1, axis=1, keepdims=True)
    is1 = t1 == i1n
    masked = jnp.where(is1, ninf, logits)
    m2 = jnp.max(masked, axis=1, keepdims=True)
    t2 = jnp.where(masked == m2, ncol, ninf)
    i2n = jnp.max(t2, axis=1, keepdims=True)
    is2 = t2 == i2n
    e2 = jnp.exp(m2 - m1)
    g1 = 1.0 / (1.0 + e2)
    g2 = e2 * g1
    out_ref[...] = jnp.where(is1, g1, jnp.where(is2, g2, 0.0))


def kernel(query_repr, W1, b1, W2, b2):
    b1r = b1.reshape(1, H2)
    b2r = b2.reshape(1, E)
    grid = (B // TB,)
    return pl.pallas_call(
        _gating_kernel,
        grid=grid,
        in_specs=[
            pl.BlockSpec((TB, H), lambda i: (i, 0)),
            pl.BlockSpec((H, H2), lambda i: (0, 0)),
            pl.BlockSpec((1, H2), lambda i: (0, 0)),
            pl.BlockSpec((H2, E), lambda i: (0, 0)),
            pl.BlockSpec((1, E), lambda i: (0, 0)),
        ],
        out_specs=pl.BlockSpec((TB, E), lambda i: (i, 0)),
        out_shape=jax.ShapeDtypeStruct((B, E), jnp.float32),
    )(query_repr, W1, b1r, W2, b2r)


# speedup vs baseline: 5.7590x; 1.2736x over previous
import jax
import jax.numpy as jnp
from jax.experimental import pallas as pl

B = 32768
H = 768
H2 = 384
E = 64
TB = 1024


def _gating_kernel(x_ref, w1_ref, b1_ref, w2_ref, b2_ref, out_ref):
    x = x_ref[...]
    h = jnp.dot(x, w1_ref[...], preferred_element_type=jnp.float32)
    h = jnp.maximum(h + b1_ref[...], 0.0)
    logits = jnp.dot(h, w2_ref[...], preferred_element_type=jnp.float32)
    logits = logits + b2_ref[...]

    ncol = -jax.lax.broadcasted_iota(jnp.int32, logits.shape, 1).astype(jnp.float32)
    ninf = jnp.float32(-jnp.inf)
    m1 = jnp.max(logits, axis=1, keepdims=True)
    t1 = jnp.where(logits == m1, ncol, ninf)
    i1n = jnp.max(t1, axis=1, keepdims=True)
    is1 = t1 == i1n
    masked = jnp.where(is1, ninf, logits)
    m2 = jnp.max(masked, axis=1, keepdims=True)
    t2 = jnp.where(masked == m2, ncol, ninf)
    i2n = jnp.max(t2, axis=1, keepdims=True)
    is2 = t2 == i2n
    e2 = jnp.exp(m2 - m1)
    g1 = 1.0 / (1.0 + e2)
    g2 = e2 * g1
    out_ref[...] = jnp.where(is1, g1, jnp.where(is2, g2, 0.0))


def kernel(query_repr, W1, b1, W2, b2):
    b1r = b1.reshape(1, H2)
    b2r = b2.reshape(1, E)
    grid = (B // TB,)
    return pl.pallas_call(
        _gating_kernel,
        grid=grid,
        in_specs=[
            pl.BlockSpec((TB, H), lambda i: (i, 0)),
            pl.BlockSpec((H, H2), lambda i: (0, 0)),
            pl.BlockSpec((1, H2), lambda i: (0, 0)),
            pl.BlockSpec((H2, E), lambda i: (0, 0)),
            pl.BlockSpec((1, E), lambda i: (0, 0)),
        ],
        out_specs=pl.BlockSpec((TB, E), lambda i: (i, 0)),
        out_shape=jax.ShapeDtypeStruct((B, E), jnp.float32),
    )(query_repr, W1, b1r, W2, b2r)


# speedup vs baseline: 6.5167x; 1.1316x over previous
import jax
import jax.numpy as jnp
from jax.experimental import pallas as pl

B = 32768
H = 768
H2 = 384
E = 64
TB = 2048


def _gating_kernel(x_ref, w1_ref, b1_ref, w2_ref, b2_ref, out_ref):
    x = x_ref[...]
    h = jnp.dot(x, w1_ref[...], preferred_element_type=jnp.float32)
    h = jnp.maximum(h + b1_ref[...], 0.0)
    logits = jnp.dot(h, w2_ref[...], preferred_element_type=jnp.float32)
    logits = logits + b2_ref[...]

    ncol = -jax.lax.broadcasted_iota(jnp.int32, logits.shape, 1).astype(jnp.float32)
    ninf = jnp.float32(-jnp.inf)
    m1 = jnp.max(logits, axis=1, keepdims=True)
    t1 = jnp.where(logits == m1, ncol, ninf)
    i1n = jnp.max(t1, axis=1, keepdims=True)
    is1 = t1 == i1n
    masked = jnp.where(is1, ninf, logits)
    m2 = jnp.max(masked, axis=1, keepdims=True)
    t2 = jnp.where(masked == m2, ncol, ninf)
    i2n = jnp.max(t2, axis=1, keepdims=True)
    is2 = t2 == i2n
    e2 = jnp.exp(m2 - m1)
    g1 = 1.0 / (1.0 + e2)
    g2 = e2 * g1
    out_ref[...] = jnp.where(is1, g1, jnp.where(is2, g2, 0.0))


def kernel(query_repr, W1, b1, W2, b2):
    b1r = b1.reshape(1, H2)
    b2r = b2.reshape(1, E)
    grid = (B // TB,)
    return pl.pallas_call(
        _gating_kernel,
        grid=grid,
        in_specs=[
            pl.BlockSpec((TB, H), lambda i: (i, 0)),
            pl.BlockSpec((H, H2), lambda i: (0, 0)),
            pl.BlockSpec((1, H2), lambda i: (0, 0)),
            pl.BlockSpec((H2, E), lambda i: (0, 0)),
            pl.BlockSpec((1, E), lambda i: (0, 0)),
        ],
        out_specs=pl.BlockSpec((TB, E), lambda i: (i, 0)),
        out_shape=jax.ShapeDtypeStruct((B, E), jnp.float32),
    )(query_repr, W1, b1r, W2, b2r)


# speedup vs baseline: 6.6144x; 1.0150x over previous
import jax
import jax.numpy as jnp
from jax.experimental import pallas as pl

B = 32768
H = 768
H2 = 384
E = 64
TB = 4096


def _gating_kernel(x_ref, w1_ref, b1_ref, w2_ref, b2_ref, out_ref):
    x = x_ref[...]
    h = jnp.dot(x, w1_ref[...], preferred_element_type=jnp.float32)
    h = jnp.maximum(h + b1_ref[...], 0.0)
    logits = jnp.dot(h, w2_ref[...], preferred_element_type=jnp.float32)
    logits = logits + b2_ref[...]

    ncol = -jax.lax.broadcasted_iota(jnp.int32, logits.shape, 1).astype(jnp.float32)
    ninf = jnp.float32(-jnp.inf)
    m1 = jnp.max(logits, axis=1, keepdims=True)
    t1 = jnp.where(logits == m1, ncol, ninf)
    i1n = jnp.max(t1, axis=1, keepdims=True)
    is1 = t1 == i1n
    masked = jnp.where(is1, ninf, logits)
    m2 = jnp.max(masked, axis=1, keepdims=True)
    t2 = jnp.where(masked == m2, ncol, ninf)
    i2n = jnp.max(t2, axis=1, keepdims=True)
    is2 = t2 == i2n
    e2 = jnp.exp(m2 - m1)
    g1 = 1.0 / (1.0 + e2)
    g2 = e2 * g1
    out_ref[...] = jnp.where(is1, g1, jnp.where(is2, g2, 0.0))


def kernel(query_repr, W1, b1, W2, b2):
    b1r = b1.reshape(1, H2)
    b2r = b2.reshape(1, E)
    grid = (B // TB,)
    return pl.pallas_call(
        _gating_kernel,
        grid=grid,
        in_specs=[
            pl.BlockSpec((TB, H), lambda i: (i, 0)),
            pl.BlockSpec((H, H2), lambda i: (0, 0)),
            pl.BlockSpec((1, H2), lambda i: (0, 0)),
            pl.BlockSpec((H2, E), lambda i: (0, 0)),
            pl.BlockSpec((1, E), lambda i: (0, 0)),
        ],
        out_specs=pl.BlockSpec((TB, E), lambda i: (i, 0)),
        out_shape=jax.ShapeDtypeStruct((B, E), jnp.float32),
    )(query_repr, W1, b1r, W2, b2r)


# speedup vs baseline: 6.6192x; 1.0007x over previous
import jax
import jax.numpy as jnp
from jax.experimental import pallas as pl
from jax.experimental.pallas import tpu as pltpu

B = 32768
H = 768
H2 = 384
E = 64
TB = 4096


def _gating_kernel(x_ref, w1_ref, b1_ref, w2_ref, b2_ref, out_ref):
    x = x_ref[...]
    h = jnp.dot(x, w1_ref[...], preferred_element_type=jnp.float32)
    h = jnp.maximum(h + b1_ref[...], 0.0)
    logits = jnp.dot(h, w2_ref[...], preferred_element_type=jnp.float32)
    logits = logits + b2_ref[...]

    ncol = -jax.lax.broadcasted_iota(jnp.int32, logits.shape, 1).astype(jnp.float32)
    ninf = jnp.float32(-jnp.inf)
    m1 = jnp.max(logits, axis=1, keepdims=True)
    t1 = jnp.where(logits == m1, ncol, ninf)
    i1n = jnp.max(t1, axis=1, keepdims=True)
    is1 = t1 == i1n
    masked = jnp.where(is1, ninf, logits)
    m2 = jnp.max(masked, axis=1, keepdims=True)
    t2 = jnp.where(masked == m2, ncol, ninf)
    i2n = jnp.max(t2, axis=1, keepdims=True)
    is2 = t2 == i2n
    e2 = jnp.exp(m2 - m1)
    g1 = 1.0 / (1.0 + e2)
    g2 = e2 * g1
    out_ref[...] = jnp.where(is1, g1, jnp.where(is2, g2, 0.0))


def kernel(query_repr, W1, b1, W2, b2):
    b1r = b1.reshape(1, H2)
    b2r = b2.reshape(1, E)
    grid = (B // TB,)
    return pl.pallas_call(
        _gating_kernel,
        grid=grid,
        in_specs=[
            pl.BlockSpec((TB, H), lambda i: (i, 0)),
            pl.BlockSpec((H, H2), lambda i: (0, 0)),
            pl.BlockSpec((1, H2), lambda i: (0, 0)),
            pl.BlockSpec((H2, E), lambda i: (0, 0)),
            pl.BlockSpec((1, E), lambda i: (0, 0)),
        ],
        out_specs=pl.BlockSpec((TB, E), lambda i: (i, 0)),
        out_shape=jax.ShapeDtypeStruct((B, E), jnp.float32),
        compiler_params=pltpu.CompilerParams(
            dimension_semantics=("parallel",),
        ),
    )(query_repr, W1, b1r, W2, b2r)
